# Initial kernel scaffold; baseline (speedup 1.0000x reference)
import jax
import jax.numpy as jnp
from jax.experimental import pallas as pl


def kernel(edge_attr, edge_index):
    raise NotImplementedError("write your pallas kernel here")


# speedup vs baseline: 11.2278x; 11.2278x over previous
import jax
import jax.numpy as jnp
from jax import lax
from jax.experimental import pallas as pl
from jax.experimental.pallas import tpu as pltpu
from jax.experimental.pallas import tpu_sc as plsc

N_NODES = 100000
N_EDGES = 3200000
MAX_DEG = 63

NUM_TILES = 16
LANES = 16

NPAD = 102400
SLICE = NPAD // NUM_TILES

ROW = 128
NROWS = N_EDGES // ROW
ROWS_PER_TILE = 1560
REM_ROWS = NROWS - NUM_TILES * ROWS_PER_TILE
CHUNK_ROWS = 120
N_CHUNKS = ROWS_PER_TILE // CHUNK_ROWS
LAG = 4


def _zero_slice(sid, summ, acc):
    zv = jnp.zeros((LANES,), jnp.float32)

    def body(i, _):
        base = i * (LANES * 4)
        for u in range(4):
            summ[pl.ds(base + u * LANES, LANES)] = zv
        return 0

    lax.fori_loop(0, SLICE // (LANES * 4), body, 0, unroll=False)
    pltpu.sync_copy(summ, acc.at[pl.ds(sid * SLICE, SLICE)])


def _scatter_rows(n, idx_buf, attr_buf, acc, sem):
    for j in range(LAG):
        pltpu.async_copy(attr_buf.at[j], acc.at[idx_buf.at[j]], sem, add=True)

    def body(j, _):
        pltpu.async_copy(attr_buf.at[j + LAG], acc.at[idx_buf.at[j + LAG]],
                         sem, add=True)
        pltpu.make_async_copy(attr_buf.at[j], acc.at[idx_buf.at[j]], sem).wait()
        return 0

    lax.fori_loop(0, n - LAG, body, 0, unroll=False)
    for j in range(n - LAG, n):
        pltpu.make_async_copy(attr_buf.at[j], acc.at[idx_buf.at[j]], sem).wait()


def _accumulate(sid, eidx_hbm, attr_hbm, idx_buf, attr_buf, acc, sem):
    row0 = sid * ROWS_PER_TILE

    def chunk_body(c, _):
        r = pl.multiple_of(row0 + c * CHUNK_ROWS, 8)
        pltpu.sync_copy(eidx_hbm.at[pl.ds(r, CHUNK_ROWS)], idx_buf)
        pltpu.sync_copy(attr_hbm.at[pl.ds(r, CHUNK_ROWS)], attr_buf)
        _scatter_rows(CHUNK_ROWS, idx_buf, attr_buf, acc, sem)
        return 0

    lax.fori_loop(0, N_CHUNKS, chunk_body, 0, unroll=False)

    @pl.when(sid == 0)
    def _():
        base = NUM_TILES * ROWS_PER_TILE
        pltpu.sync_copy(eidx_hbm.at[pl.ds(base, REM_ROWS)],
                        idx_buf.at[pl.ds(0, REM_ROWS)])
        pltpu.sync_copy(attr_hbm.at[pl.ds(base, REM_ROWS)],
                        attr_buf.at[pl.ds(0, REM_ROWS)])
        _scatter_rows(REM_ROWS, idx_buf, attr_buf, acc, sem)


def _interp_write(sid, acc, summ, idx2d, w2d, oidx_hbm, ow_hbm):
    nbase = sid * SLICE
    pltpu.sync_copy(acc.at[pl.ds(nbase, SLICE)], summ)

    def interp(j, _):
        for u in range(2):
            s = (j * 2 + u) * LANES
            deg = summ[pl.ds(s, LANES)]
            deg = jnp.minimum(deg, jnp.float32(MAX_DEG))
            low = deg.astype(jnp.int32)
            frac = deg - low.astype(jnp.float32)
            hasf = frac > 0.0
            high = low + jnp.where(hasf, 1, 0)
            w_low = jnp.where(hasf, 1.0 - frac, jnp.float32(1.0))
            w_low = jnp.where(low == 0, jnp.float32(0.0), w_low)
            pos0 = (lax.iota(jnp.int32, LANES) + s) * 2
            pos1 = pos0 + 1
            plsc.store_scatter(idx2d, [pos0], low)
            plsc.store_scatter(idx2d, [pos1], high)
            plsc.store_scatter(w2d, [pos0], w_low)
            plsc.store_scatter(w2d, [pos1], frac)
        return 0

    lax.fori_loop(0, SLICE // (LANES * 2), interp, 0, unroll=False)
    pltpu.sync_copy(idx2d, oidx_hbm.at[pl.ds(2 * nbase, 2 * SLICE)])
    pltpu.sync_copy(w2d, ow_hbm.at[pl.ds(2 * nbase, 2 * SLICE)])


def _degree_pipeline(sid, eidx_hbm, attr_hbm, oidx_hbm, ow_hbm,
                     idx_buf, attr_buf, summ, idx2d, w2d, acc, sem):
    _zero_slice(sid, summ, acc)
    plsc.subcore_barrier()
    _accumulate(sid, eidx_hbm, attr_hbm, idx_buf, attr_buf, acc, sem)
    plsc.subcore_barrier()
    _interp_write(sid, acc, summ, idx2d, w2d, oidx_hbm, ow_hbm)


def _sc_body(attr_hbm, dst_hbm, src_hbm,
             in_idx_hbm, in_w_hbm, out_idx_hbm, out_w_hbm,
             idx_buf, attr_buf, summ, idx2d, w2d, acc, sem):
    cid = lax.axis_index("c")
    sid = lax.axis_index("s")

    @pl.when(cid == 0)
    def _():
        _degree_pipeline(sid, dst_hbm, attr_hbm, in_idx_hbm, in_w_hbm,
                         idx_buf, attr_buf, summ, idx2d, w2d, acc, sem)

    @pl.when(cid == 1)
    def _():
        _degree_pipeline(sid, src_hbm, attr_hbm, out_idx_hbm, out_w_hbm,
                         idx_buf, attr_buf, summ, idx2d, w2d, acc, sem)


@jax.jit
def kernel(edge_attr, edge_index):
    dst = edge_index[1].reshape(NROWS, ROW)
    src = edge_index[0].reshape(NROWS, ROW)
    attr = edge_attr.reshape(NROWS, ROW)

    mesh = plsc.VectorSubcoreMesh(core_axis_name="c", subcore_axis_name="s")
    run = pl.kernel(
        _sc_body,
        out_type=[
            jax.ShapeDtypeStruct((NPAD * 2,), jnp.int32),
            jax.ShapeDtypeStruct((NPAD * 2,), jnp.float32),
            jax.ShapeDtypeStruct((NPAD * 2,), jnp.int32),
            jax.ShapeDtypeStruct((NPAD * 2,), jnp.float32),
        ],
        mesh=mesh,
        compiler_params=pltpu.CompilerParams(needs_layout_passes=False),
        scratch_types=[
            pltpu.VMEM((CHUNK_ROWS, ROW), jnp.int32),
            pltpu.VMEM((CHUNK_ROWS, ROW), jnp.float32),
            pltpu.VMEM((SLICE,), jnp.float32),
            pltpu.VMEM((SLICE * 2,), jnp.int32),
            pltpu.VMEM((SLICE * 2,), jnp.float32),
            pltpu.VMEM_SHARED((NPAD,), jnp.float32),
            pltpu.SemaphoreType.DMA,
        ],
    )
    in_idx, in_w, out_idx, out_w = run(attr, dst, src)
    return (in_idx.reshape(NPAD, 2)[:N_NODES],
            in_w.reshape(NPAD, 2)[:N_NODES],
            out_idx.reshape(NPAD, 2)[:N_NODES],
            out_w.reshape(NPAD, 2)[:N_NODES])


# speedup vs baseline: 32.6284x; 2.9060x over previous
import jax
import jax.numpy as jnp
from jax import lax
from jax.experimental import pallas as pl
from jax.experimental.pallas import tpu as pltpu
from jax.experimental.pallas import tpu_sc as plsc

N_NODES = 100000
N_EDGES = 3200000
MAX_DEG = 63

NUM_TILES = 16
LANES = 16

NPAD = 102400
SLICE = NPAD // NUM_TILES
HALF = SLICE // 2
ICHUNK = 1280

EDGES_PER_TILE = N_EDGES // NUM_TILES
EC = 4000
N_CHUNKS = EDGES_PER_TILE // EC


def _zero_acc(acc):
    zv = jnp.zeros((LANES,), jnp.float32)

    def body(i, _):
        base = i * (LANES * 8)
        for u in range(8):
            acc[pl.ds(base + u * LANES, LANES)] = zv
        return 0

    lax.fori_loop(0, NPAD // (LANES * 8), body, 0, unroll=False)


def _start_load(c, ebase, eidx_hbm, attr_hbm, ibuf, abuf, sem):
    start = pl.multiple_of(ebase + c * EC, 8)
    pltpu.async_copy(eidx_hbm.at[pl.ds(start, EC)], ibuf, sem)
    pltpu.async_copy(attr_hbm.at[pl.ds(start, EC)], abuf, sem)


def _wait_load(c, ebase, eidx_hbm, attr_hbm, ibuf, abuf, sem):
    start = pl.multiple_of(ebase + c * EC, 8)
    pltpu.make_async_copy(eidx_hbm.at[pl.ds(start, EC)], ibuf, sem).wait()
    pltpu.make_async_copy(attr_hbm.at[pl.ds(start, EC)], abuf, sem).wait()


def _scatter_chunk(ibuf, abuf, acc):
    def g(j, _):
        for u in range(5):
            off = (j * 5 + u) * LANES
            iv = ibuf[pl.ds(off, LANES)]
            av = abuf[pl.ds(off, LANES)]
            plsc.addupdate_scatter(acc, [iv], av)
        return 0

    lax.fori_loop(0, EC // (LANES * 5), g, 0, unroll=False)


def _accumulate(sid, eidx_hbm, attr_hbm, ibuf0, abuf0, ibuf1, abuf1,
                acc, sem0, sem1):
    ebase = sid * EDGES_PER_TILE
    _start_load(0, ebase, eidx_hbm, attr_hbm, ibuf0, abuf0, sem0)

    def pair(k, _):
        c0 = k * 2
        _start_load(c0 + 1, ebase, eidx_hbm, attr_hbm, ibuf1, abuf1, sem1)
        _wait_load(c0, ebase, eidx_hbm, attr_hbm, ibuf0, abuf0, sem0)
        _scatter_chunk(ibuf0, abuf0, acc)

        @pl.when(c0 + 2 < N_CHUNKS)
        def _():
            _start_load(c0 + 2, ebase, eidx_hbm, attr_hbm, ibuf0, abuf0, sem0)

        _wait_load(c0 + 1, ebase, eidx_hbm, attr_hbm, ibuf1, abuf1, sem1)
        _scatter_chunk(ibuf1, abuf1, acc)
        return 0

    lax.fori_loop(0, N_CHUNKS // 2, pair, 0, unroll=False)


def _reduce(sid, acc, staging, tmp):
    own = sid * SLICE
    for r in range(1, NUM_TILES):
        o = lax.rem(sid + r, NUM_TILES)
        for q in range(2):
            src = pl.multiple_of(o * SLICE + q * HALF, 8)
            pltpu.sync_copy(acc.at[pl.ds(src, HALF)], staging.at[o])
            plsc.subcore_barrier()
            pltpu.sync_copy(staging.at[sid], tmp)

            def addb(j, _):
                for u in range(4):
                    s = (j * 4 + u) * LANES
                    d = own + q * HALF + s
                    acc[pl.ds(d, LANES)] = acc[pl.ds(d, LANES)] + tmp[pl.ds(s, LANES)]
                return 0

            lax.fori_loop(0, HALF // (LANES * 4), addb, 0, unroll=False)
            plsc.subcore_barrier()


def _interp_write(sid, acc, lo_i, hi_i, lo_w, hi_w, oidx_hbm, ow_hbm):
    own = sid * SLICE
    for k in range(SLICE // ICHUNK):
        nbase = own + k * ICHUNK

        def interp(j, _):
            for u in range(2):
                s = (j * 2 + u) * LANES
                deg = acc[pl.ds(nbase + s, LANES)]
                deg = jnp.minimum(deg, jnp.float32(MAX_DEG))
                low = deg.astype(jnp.int32)
                frac = deg - low.astype(jnp.float32)
                hasf = frac > 0.0
                high = low + jnp.where(hasf, 1, 0)
                w_low = jnp.where(hasf, 1.0 - frac, jnp.float32(1.0))
                w_low = jnp.where(low == 0, jnp.float32(0.0), w_low)
                lo_i[pl.ds(s, LANES)] = low
                hi_i[pl.ds(s, LANES)] = high
                lo_w[pl.ds(s, LANES)] = w_low
                hi_w[pl.ds(s, LANES)] = frac
            return 0

        lax.fori_loop(0, ICHUNK // (LANES * 2), interp, 0, unroll=False)
        pltpu.sync_copy(lo_i, oidx_hbm.at[pl.ds(nbase, ICHUNK)])
        pltpu.sync_copy(hi_i, oidx_hbm.at[pl.ds(NPAD + nbase, ICHUNK)])
        pltpu.sync_copy(lo_w, ow_hbm.at[pl.ds(nbase, ICHUNK)])
        pltpu.sync_copy(hi_w, ow_hbm.at[pl.ds(NPAD + nbase, ICHUNK)])


def _degree_pipeline(sid, eidx_hbm, attr_hbm, oidx_hbm, ow_hbm,
                     ibuf0, abuf0, ibuf1, abuf1, tmp,
                     lo_i, hi_i, lo_w, hi_w, acc, staging, sem0, sem1):
    _zero_acc(acc)
    _accumulate(sid, eidx_hbm, attr_hbm, ibuf0, abuf0, ibuf1, abuf1,
                acc, sem0, sem1)
    plsc.subcore_barrier()
    _reduce(sid, acc, staging, tmp)
    _interp_write(sid, acc, lo_i, hi_i, lo_w, hi_w, oidx_hbm, ow_hbm)


def _sc_body(attr_hbm, dst_hbm, src_hbm,
             in_idx_hbm, in_w_hbm, out_idx_hbm, out_w_hbm,
             ibuf0, abuf0, ibuf1, abuf1, tmp,
             lo_i, hi_i, lo_w, hi_w, acc, staging, sem0, sem1):
    cid = lax.axis_index("c")
    sid = lax.axis_index("s")

    @pl.when(cid == 0)
    def _():
        _degree_pipeline(sid, dst_hbm, attr_hbm, in_idx_hbm, in_w_hbm,
                         ibuf0, abuf0, ibuf1, abuf1, tmp,
                         lo_i, hi_i, lo_w, hi_w, acc, staging, sem0, sem1)

    @pl.when(cid == 1)
    def _():
        _degree_pipeline(sid, src_hbm, attr_hbm, out_idx_hbm, out_w_hbm,
                         ibuf0, abuf0, ibuf1, abuf1, tmp,
                         lo_i, hi_i, lo_w, hi_w, acc, staging, sem0, sem1)


@jax.jit
def kernel(edge_attr, edge_index):
    dst = edge_index[1]
    src = edge_index[0]

    mesh = plsc.VectorSubcoreMesh(core_axis_name="c", subcore_axis_name="s")
    run = pl.kernel(
        _sc_body,
        out_type=[
            jax.ShapeDtypeStruct((NPAD * 2,), jnp.int32),
            jax.ShapeDtypeStruct((NPAD * 2,), jnp.float32),
            jax.ShapeDtypeStruct((NPAD * 2,), jnp.int32),
            jax.ShapeDtypeStruct((NPAD * 2,), jnp.float32),
        ],
        mesh=mesh,
        compiler_params=pltpu.CompilerParams(needs_layout_passes=False),
        scratch_types=[
            pltpu.VMEM((EC,), jnp.int32),
            pltpu.VMEM((EC,), jnp.float32),
            pltpu.VMEM((EC,), jnp.int32),
            pltpu.VMEM((EC,), jnp.float32),
            pltpu.VMEM((HALF,), jnp.float32),
            pltpu.VMEM((ICHUNK,), jnp.int32),
            pltpu.VMEM((ICHUNK,), jnp.int32),
            pltpu.VMEM((ICHUNK,), jnp.float32),
            pltpu.VMEM((ICHUNK,), jnp.float32),
            pltpu.VMEM((NPAD,), jnp.float32),
            pltpu.VMEM_SHARED((NUM_TILES, HALF), jnp.float32),
            pltpu.SemaphoreType.DMA,
            pltpu.SemaphoreType.DMA,
        ],
    )
    in_idx, in_w, out_idx, out_w = run(edge_attr, dst, src)

    def planes_to_pairs(flat):
        return jnp.concatenate(
            [flat[:N_NODES, None], flat[NPAD:NPAD + N_NODES, None]], axis=1)

    return (planes_to_pairs(in_idx), planes_to_pairs(in_w),
            planes_to_pairs(out_idx), planes_to_pairs(out_w))


# speedup vs baseline: 45.1885x; 1.3849x over previous
import jax
import jax.numpy as jnp
from jax import lax
from jax.experimental import pallas as pl
from jax.experimental.pallas import tpu as pltpu
from jax.experimental.pallas import tpu_sc as plsc

N_NODES = 100000
N_EDGES = 3200000
MAX_DEG = 63

NUM_TILES = 16
LANES = 16

NPAD = 102400
SLICE = NPAD // NUM_TILES
HALF = SLICE // 2
ICHUNK = 1280

EDGES_PER_TILE = N_EDGES // NUM_TILES
EC = 4000
N_CHUNKS = EDGES_PER_TILE // EC


def _zero_acc(acc):
    zv = jnp.zeros((LANES,), jnp.float32)

    def body(i, _):
        base = i * (LANES * 8)
        for u in range(8):
            acc[pl.ds(base + u * LANES, LANES)] = zv
        return 0

    lax.fori_loop(0, NPAD // (LANES * 8), body, 0, unroll=False)


def _start_load(c, ebase, eidx_hbm, attr_hbm, ibuf, abuf, sem):
    start = pl.multiple_of(ebase + c * EC, 8)
    pltpu.async_copy(eidx_hbm.at[pl.ds(start, EC)], ibuf, sem)
    pltpu.async_copy(attr_hbm.at[pl.ds(start, EC)], abuf, sem)


def _wait_load(c, ebase, eidx_hbm, attr_hbm, ibuf, abuf, sem):
    start = pl.multiple_of(ebase + c * EC, 8)
    pltpu.make_async_copy(eidx_hbm.at[pl.ds(start, EC)], ibuf, sem).wait()
    pltpu.make_async_copy(attr_hbm.at[pl.ds(start, EC)], abuf, sem).wait()


def _scatter_chunk(ibuf, abuf, acc):
    UNR = 10

    def g(j, _):
        base = j * UNR * LANES
        ivs = [ibuf[pl.ds(base + u * LANES, LANES)] for u in range(UNR)]
        avs = [abuf[pl.ds(base + u * LANES, LANES)] for u in range(UNR)]
        for u in range(UNR):
            plsc.addupdate_scatter(acc, [ivs[u]], avs[u])
        return 0

    lax.fori_loop(0, EC // (LANES * UNR), g, 0, unroll=False)


def _accumulate(sid, eidx_hbm, attr_hbm, ibuf0, abuf0, ibuf1, abuf1,
                acc, sem0, sem1):
    ebase = sid * EDGES_PER_TILE
    _start_load(0, ebase, eidx_hbm, attr_hbm, ibuf0, abuf0, sem0)

    def pair(k, _):
        c0 = k * 2
        _start_load(c0 + 1, ebase, eidx_hbm, attr_hbm, ibuf1, abuf1, sem1)
        _wait_load(c0, ebase, eidx_hbm, attr_hbm, ibuf0, abuf0, sem0)
        _scatter_chunk(ibuf0, abuf0, acc)

        @pl.when(c0 + 2 < N_CHUNKS)
        def _():
            _start_load(c0 + 2, ebase, eidx_hbm, attr_hbm, ibuf0, abuf0, sem0)

        _wait_load(c0 + 1, ebase, eidx_hbm, attr_hbm, ibuf1, abuf1, sem1)
        _scatter_chunk(ibuf1, abuf1, acc)
        return 0

    lax.fori_loop(0, N_CHUNKS // 2, pair, 0, unroll=False)


def _reduce(sid, acc, staging, tmp):
    own = sid * SLICE
    for r in range(1, NUM_TILES):
        o = lax.rem(sid + r, NUM_TILES)
        for q in range(2):
            src = pl.multiple_of(o * SLICE + q * HALF, 8)
            pltpu.sync_copy(acc.at[pl.ds(src, HALF)], staging.at[o])
            plsc.subcore_barrier()
            pltpu.sync_copy(staging.at[sid], tmp)

            def addb(j, _):
                s0 = j * 8 * LANES
                d0 = own + q * HALF + s0
                axs = [acc[pl.ds(d0 + u * LANES, LANES)] for u in range(8)]
                txs = [tmp[pl.ds(s0 + u * LANES, LANES)] for u in range(8)]
                for u in range(8):
                    acc[pl.ds(d0 + u * LANES, LANES)] = axs[u] + txs[u]
                return 0

            lax.fori_loop(0, HALF // (LANES * 8), addb, 0, unroll=False)
            plsc.subcore_barrier()


def _interp_write(sid, acc, lo_i, hi_i, lo_w, hi_w, oidx_hbm, ow_hbm):
    own = sid * SLICE
    for k in range(SLICE // ICHUNK):
        nbase = own + k * ICHUNK

        def interp(j, _):
            for u in range(2):
                s = (j * 2 + u) * LANES
                deg = acc[pl.ds(nbase + s, LANES)]
                deg = jnp.minimum(deg, jnp.float32(MAX_DEG))
                low = deg.astype(jnp.int32)
                frac = deg - low.astype(jnp.float32)
                hasf = frac > 0.0
                high = low + jnp.where(hasf, 1, 0)
                w_low = jnp.where(hasf, 1.0 - frac, jnp.float32(1.0))
                w_low = jnp.where(low == 0, jnp.float32(0.0), w_low)
                lo_i[pl.ds(s, LANES)] = low
                hi_i[pl.ds(s, LANES)] = high
                lo_w[pl.ds(s, LANES)] = w_low
                hi_w[pl.ds(s, LANES)] = frac
            return 0

        lax.fori_loop(0, ICHUNK // (LANES * 2), interp, 0, unroll=False)
        pltpu.sync_copy(lo_i, oidx_hbm.at[pl.ds(nbase, ICHUNK)])
        pltpu.sync_copy(hi_i, oidx_hbm.at[pl.ds(NPAD + nbase, ICHUNK)])
        pltpu.sync_copy(lo_w, ow_hbm.at[pl.ds(nbase, ICHUNK)])
        pltpu.sync_copy(hi_w, ow_hbm.at[pl.ds(NPAD + nbase, ICHUNK)])


def _degree_pipeline(sid, eidx_hbm, attr_hbm, oidx_hbm, ow_hbm,
                     ibuf0, abuf0, ibuf1, abuf1, tmp,
                     lo_i, hi_i, lo_w, hi_w, acc, staging, sem0, sem1):
    _zero_acc(acc)
    _accumulate(sid, eidx_hbm, attr_hbm, ibuf0, abuf0, ibuf1, abuf1,
                acc, sem0, sem1)
    plsc.subcore_barrier()
    _reduce(sid, acc, staging, tmp)
    _interp_write(sid, acc, lo_i, hi_i, lo_w, hi_w, oidx_hbm, ow_hbm)


def _sc_body(attr_hbm, dst_hbm, src_hbm,
             in_idx_hbm, in_w_hbm, out_idx_hbm, out_w_hbm,
             ibuf0, abuf0, ibuf1, abuf1, tmp,
             lo_i, hi_i, lo_w, hi_w, acc, staging, sem0, sem1):
    cid = lax.axis_index("c")
    sid = lax.axis_index("s")

    @pl.when(cid == 0)
    def _():
        _degree_pipeline(sid, dst_hbm, attr_hbm, in_idx_hbm, in_w_hbm,
                         ibuf0, abuf0, ibuf1, abuf1, tmp,
                         lo_i, hi_i, lo_w, hi_w, acc, staging, sem0, sem1)

    @pl.when(cid == 1)
    def _():
        _degree_pipeline(sid, src_hbm, attr_hbm, out_idx_hbm, out_w_hbm,
                         ibuf0, abuf0, ibuf1, abuf1, tmp,
                         lo_i, hi_i, lo_w, hi_w, acc, staging, sem0, sem1)


@jax.jit
def kernel(edge_attr, edge_index):
    dst = edge_index[1]
    src = edge_index[0]

    mesh = plsc.VectorSubcoreMesh(core_axis_name="c", subcore_axis_name="s")
    run = pl.kernel(
        _sc_body,
        out_type=[
            jax.ShapeDtypeStruct((NPAD * 2,), jnp.int32),
            jax.ShapeDtypeStruct((NPAD * 2,), jnp.float32),
            jax.ShapeDtypeStruct((NPAD * 2,), jnp.int32),
            jax.ShapeDtypeStruct((NPAD * 2,), jnp.float32),
        ],
        mesh=mesh,
        compiler_params=pltpu.CompilerParams(needs_layout_passes=False),
        scratch_types=[
            pltpu.VMEM((EC,), jnp.int32),
            pltpu.VMEM((EC,), jnp.float32),
            pltpu.VMEM((EC,), jnp.int32),
            pltpu.VMEM((EC,), jnp.float32),
            pltpu.VMEM((HALF,), jnp.float32),
            pltpu.VMEM((ICHUNK,), jnp.int32),
            pltpu.VMEM((ICHUNK,), jnp.int32),
            pltpu.VMEM((ICHUNK,), jnp.float32),
            pltpu.VMEM((ICHUNK,), jnp.float32),
            pltpu.VMEM((NPAD,), jnp.float32),
            pltpu.VMEM_SHARED((NUM_TILES, HALF), jnp.float32),
            pltpu.SemaphoreType.DMA,
            pltpu.SemaphoreType.DMA,
        ],
    )
    in_idx, in_w, out_idx, out_w = run(edge_attr, dst, src)

    def planes_to_pairs(flat):
        return jnp.concatenate(
            [flat[:N_NODES, None], flat[NPAD:NPAD + N_NODES, None]], axis=1)

    return (planes_to_pairs(in_idx), planes_to_pairs(in_w),
            planes_to_pairs(out_idx), planes_to_pairs(out_w))
